# Initial kernel scaffold; baseline (speedup 1.0000x reference)
import jax
import jax.numpy as jnp
from jax.experimental import pallas as pl


def kernel(x, router_W, router_b, W1, b1, W2, b2):
    raise NotImplementedError("write your pallas kernel here")


# speedup vs baseline: 1.2424x; 1.2424x over previous
import functools

import jax
import jax.numpy as jnp
from jax.experimental import pallas as pl
from jax.experimental.pallas import tpu as pltpu

B, T, DIM = 2, 2048, 768
E, K, HID = 8, 2, 2048
INNER = 2 * HID
N = B * T
TILE = 512
NT = N // TILE


def _router_body(x_ref, rw_ref, rb_ref, comb_ref):
    logits = jax.lax.dot_general(
        x_ref[...], rw_ref[...],
        (((1,), (1,)), ((), ())),
        preferred_element_type=jnp.float32,
    ) + rb_ref[...]
    idx = jax.lax.broadcasted_iota(jnp.int32, (N, E), 1)
    m1 = jnp.max(logits, axis=1, keepdims=True)
    a1 = jnp.min(jnp.where(logits == m1, idx, E), axis=1, keepdims=True)
    l2 = jnp.where(idx == a1, -jnp.inf, logits)
    m2 = jnp.max(l2, axis=1, keepdims=True)
    a2 = jnp.min(jnp.where(l2 == m2, idx, E), axis=1, keepdims=True)
    w1 = jax.nn.sigmoid(m1 - m2)
    w2 = 1.0 - w1
    comb_ref[...] = jnp.where(idx == a1, w1, 0.0) + jnp.where(idx == a2, w2, 0.0)


def _expert_body(xb_ref, comb_ref, w1_ref, b1_ref, w2_ref, b2_ref, out_ref):
    e = pl.program_id(0)
    t = pl.program_id(1)
    rows = pl.ds(t * TILE, TILE)
    xt = xb_ref[rows, :]
    y = jnp.zeros((TILE, DIM), jnp.float32)
    CH = 1024
    for j0 in range(0, HID, CH):
        a = jnp.dot(xt, w1_ref[0, :, j0:j0 + CH],
                    preferred_element_type=jnp.float32) + b1_ref[0, 0, j0:j0 + CH]
        g = jnp.dot(xt, w1_ref[0, :, HID + j0:HID + j0 + CH],
                    preferred_element_type=jnp.float32) + b1_ref[0, 0, HID + j0:HID + j0 + CH]
        act = (a * jax.nn.sigmoid(a)) * g
        y = y + jnp.dot(act.astype(jnp.bfloat16), w2_ref[0, j0:j0 + CH, :],
                        preferred_element_type=jnp.float32)
    y = y + b2_ref[0, 0, :]
    cb = comb_ref[rows, :]
    sel = jax.lax.broadcasted_iota(jnp.int32, (TILE, E), 1) == e
    c = jnp.sum(jnp.where(sel, cb, 0.0), axis=1, keepdims=True)
    contrib = c * y

    @pl.when(e == 0)
    def _init():
        out_ref[rows, :] = contrib

    @pl.when(e != 0)
    def _acc():
        out_ref[rows, :] = out_ref[rows, :] + contrib


@jax.jit
def kernel(x, router_W, router_b, W1, b1, W2, b2):
    xf = x.reshape(N, DIM)

    comb = pl.pallas_call(
        _router_body,
        out_shape=jax.ShapeDtypeStruct((N, E), jnp.float32),
        in_specs=[
            pl.BlockSpec((N, DIM), lambda: (0, 0)),
            pl.BlockSpec((E, DIM), lambda: (0, 0)),
            pl.BlockSpec((E,), lambda: (0,)),
        ],
        out_specs=pl.BlockSpec((N, E), lambda: (0, 0)),
    )(xf, router_W, router_b)

    xb = xf.astype(jnp.bfloat16)
    w1b = W1.astype(jnp.bfloat16)
    w2b = W2.astype(jnp.bfloat16)

    out = pl.pallas_call(
        _expert_body,
        grid=(E, NT),
        out_shape=jax.ShapeDtypeStruct((N, DIM), jnp.float32),
        in_specs=[
            pl.BlockSpec((N, DIM), lambda e, t: (0, 0)),
            pl.BlockSpec((N, E), lambda e, t: (0, 0)),
            pl.BlockSpec((1, DIM, INNER), lambda e, t: (e, 0, 0)),
            pl.BlockSpec((1, 1, INNER), lambda e, t: (e, 0, 0)),
            pl.BlockSpec((1, HID, DIM), lambda e, t: (e, 0, 0)),
            pl.BlockSpec((1, 1, DIM), lambda e, t: (e, 0, 0)),
        ],
        out_specs=pl.BlockSpec((N, DIM), lambda e, t: (0, 0)),
        compiler_params=pltpu.CompilerParams(
            dimension_semantics=("arbitrary", "arbitrary"),
        ),
    )(xb, comb, w1b, b1.reshape(E, 1, INNER), w2b, b2.reshape(E, 1, DIM))

    return out.reshape(B, T, DIM)


# speedup vs baseline: 1.2496x; 1.0058x over previous
import functools

import jax
import jax.numpy as jnp
from jax import lax
from jax.experimental import pallas as pl
from jax.experimental.pallas import tpu as pltpu
from jax.experimental.pallas import tpu_sc as plsc

B, T, DIM = 2, 2048, 768
E, K, HID = 8, 2, 2048
INNER = 2 * HID
N = B * T
P = N * K
TM = 256
PADTOT = P + E * TM
GT = PADTOT // TM
NSUB = 16
CP = P // NSUB
SLOTW = PADTOT // NSUB
NW2 = 32
RW = P // NW2
GCH = 64


def _router_body(x_ref, rw_ref, rb_ref, eid_ref, w_ref):
    logits = jax.lax.dot_general(
        x_ref[...], rw_ref[...], (((1,), (1,)), ((), ())),
        preferred_element_type=jnp.float32,
    ) + rb_ref[...]
    idx = jax.lax.broadcasted_iota(jnp.int32, (N, E), 1)
    m1 = jnp.max(logits, axis=1, keepdims=True)
    a1 = jnp.min(jnp.where(logits == m1, idx, E), axis=1, keepdims=True)
    l2 = jnp.where(idx == a1, -jnp.inf, logits)
    m2 = jnp.max(l2, axis=1, keepdims=True)
    a2 = jnp.min(jnp.where(l2 == m2, idx, E), axis=1, keepdims=True)
    w1 = jax.nn.sigmoid(m1 - m2)
    eid_ref[...] = jnp.concatenate([a1, a2], axis=1)
    w_ref[...] = jnp.concatenate([w1, 1.0 - w1], axis=1)


def _router(xf, router_W, router_b):
    return pl.pallas_call(
        _router_body,
        out_shape=(
            jax.ShapeDtypeStruct((N, K), jnp.int32),
            jax.ShapeDtypeStruct((N, K), jnp.float32),
        ),
        in_specs=[
            pl.BlockSpec((N, DIM), lambda: (0, 0)),
            pl.BlockSpec((E, DIM), lambda: (0, 0)),
            pl.BlockSpec((E,), lambda: (0,)),
        ],
        out_specs=(
            pl.BlockSpec((N, K), lambda: (0, 0)),
            pl.BlockSpec((N, K), lambda: (0, 0)),
        ),
    )(xf, router_W, router_b)


@functools.cache
def _mesh():
    return plsc.VectorSubcoreMesh(core_axis_name="c", subcore_axis_name="s",
                                  num_cores=2, num_subcores=NSUB)


def _lane_scalar(vec, e):
    lanes = lax.iota(jnp.int32, 16)
    return jnp.sum(jnp.where(lanes == e, vec, 0))


def _sc_dispatch_body(eids_hbm, wts_hbm, x_hbm,
                      xdisp_hbm, wslot_hbm, invpos_hbm, gids_hbm,
                      eid_v, wts_v, pos_v, hist_v, allh_v,
                      tokbuf, wbuf, tmp_tok, tmp_w, macc_tok, macc_w,
                      idx_v, rows_v, gids_v,
                      sh_hist, sh_tok, sh_w, sem):
    c = lax.axis_index("c")
    w = lax.axis_index("s")
    lanes = lax.iota(jnp.int32, 16)

    @pl.when(c == 0)
    def _core0():
        base_pair = w * CP
        pltpu.sync_copy(eids_hbm.at[pl.ds(base_pair, CP)], eid_v)
        pltpu.sync_copy(wts_hbm.at[pl.ds(base_pair, CP)], wts_v)

        def hist_step(i, h):
            v = eid_v[pl.ds(i * 16, 16)]
            return tuple(h[e] + jnp.sum(jnp.where(v == e, 1, 0)) for e in range(E))

        hist = lax.fori_loop(0, CP // 16, hist_step,
                             tuple(jnp.int32(0) for _ in range(E)))
        hvec = jnp.zeros((16,), jnp.int32)
        for e in range(E):
            hvec = jnp.where(lanes == e, hist[e], hvec)
        hist_v[...] = hvec
        pltpu.sync_copy(hist_v, sh_hist.at[w])
        plsc.subcore_barrier()

        pltpu.sync_copy(sh_hist, allh_v)

        def sum_step(i, carry):
            tot, pre = carry
            row = allh_v[i, :]
            return tot + row, pre + jnp.where(i < w, row, 0)

        tot, pre = lax.fori_loop(
            0, NSUB, sum_step,
            (jnp.zeros((16,), jnp.int32), jnp.zeros((16,), jnp.int32)))
        padcnt = ((tot + (TM - 1)) >> 8) << 8
        incl = plsc.cumsum(padcnt)
        padoff = incl - padcnt
        base_vec = padoff + pre
        bases = [_lane_scalar(base_vec, e) for e in range(E)]

        @pl.when(w == 0)
        def _gids():
            incl_s = [_lane_scalar(incl, e) for e in range(E)]
            for j in range(3):
                ts = (lax.iota(jnp.int32, 16) + 16 * j) * TM
                g = jnp.zeros((16,), jnp.int32)
                for e in range(E):
                    g = g + jnp.where(ts >= incl_s[e], 1, 0)
                gids_v[pl.ds(j * 16, 16)] = jnp.minimum(g, E - 1)
            pltpu.sync_copy(gids_v, gids_hbm)

        def zero_step(i, _):
            tokbuf[pl.ds(i * 16, 16)] = jnp.zeros((16,), jnp.int32)
            wbuf[pl.ds(i * 16, 16)] = jnp.zeros((16,), jnp.float32)
            return 0

        lax.fori_loop(0, PADTOT // 16, zero_step, 0)

        def scan_step(i, run):
            v = eid_v[pl.ds(i * 16, 16)]
            wv = wts_v[pl.ds(i * 16, 16)]
            pos = jnp.zeros((16,), jnp.int32)
            new_run = []
            for e in range(E):
                m = v == e
                mi = m.astype(jnp.int32)
                excl = plsc.cumsum(mi) - mi
                pos = jnp.where(m, run[e] + excl, pos)
                new_run.append(run[e] + jnp.sum(mi))
            pos_v[pl.ds(i * 16, 16)] = pos
            toks = ((base_pair + i * 16 + lanes) >> 1) + 1
            plsc.store_scatter(tokbuf, [pos], toks)
            plsc.store_scatter(wbuf, [pos], wv)
            return tuple(new_run)

        lax.fori_loop(0, CP // 16, scan_step, tuple(bases))
        pltpu.sync_copy(pos_v, invpos_hbm.at[pl.ds(base_pair, CP)])

        pltpu.sync_copy(tokbuf, sh_tok.at[w])
        pltpu.sync_copy(wbuf, sh_w.at[w])
        plsc.subcore_barrier()

        s0 = w * SLOTW

        def zero2_step(i, _):
            macc_tok[pl.ds(i * 16, 16)] = jnp.zeros((16,), jnp.int32)
            macc_w[pl.ds(i * 16, 16)] = jnp.zeros((16,), jnp.float32)
            return 0

        lax.fori_loop(0, SLOTW // 16, zero2_step, 0)

        def merge_step(i, _):
            pltpu.sync_copy(sh_tok.at[i, pl.ds(s0, SLOTW)], tmp_tok)
            pltpu.sync_copy(sh_w.at[i, pl.ds(s0, SLOTW)], tmp_w)

            def add_step(j, _):
                sl = pl.ds(j * 16, 16)
                macc_tok[sl] = macc_tok[sl] + tmp_tok[sl]
                macc_w[sl] = macc_w[sl] + tmp_w[sl]
                return 0

            lax.fori_loop(0, SLOTW // 16, add_step, 0)
            return 0

        lax.fori_loop(0, NSUB, merge_step, 0)
        pltpu.sync_copy(macc_w, wslot_hbm.at[pl.ds(s0, SLOTW)])

        def gather_step(g, _):
            def stage_step(j, _):
                t = macc_tok[pl.ds(g * GCH + j * 16, 16)]
                idx_v[pl.ds(j * 16, 16)] = jnp.maximum(t - 1, 0)
                return 0

            lax.fori_loop(0, GCH // 16, stage_step, 0)
            pltpu.async_copy(x_hbm.at[idx_v], rows_v, sem).wait()
            pltpu.sync_copy(rows_v, xdisp_hbm.at[pl.ds(s0 + g * GCH, GCH)])
            return 0

        lax.fori_loop(0, SLOTW // GCH, gather_step, 0)


@functools.cache
def _sc_dispatch():
    return pl.kernel(
        _sc_dispatch_body,
        out_type=(
            jax.ShapeDtypeStruct((PADTOT, DIM), jnp.float32),
            jax.ShapeDtypeStruct((PADTOT,), jnp.float32),
            jax.ShapeDtypeStruct((P,), jnp.int32),
            jax.ShapeDtypeStruct((48,), jnp.int32),
        ),
        mesh=_mesh(),
        compiler_params=pltpu.CompilerParams(needs_layout_passes=False),
        scratch_types=[
            pltpu.VMEM((CP,), jnp.int32),
            pltpu.VMEM((CP,), jnp.float32),
            pltpu.VMEM((CP,), jnp.int32),
            pltpu.VMEM((16,), jnp.int32),
            pltpu.VMEM((NSUB, 16), jnp.int32),
            pltpu.VMEM((PADTOT,), jnp.int32),
            pltpu.VMEM((PADTOT,), jnp.float32),
            pltpu.VMEM((SLOTW,), jnp.int32),
            pltpu.VMEM((SLOTW,), jnp.float32),
            pltpu.VMEM((SLOTW,), jnp.int32),
            pltpu.VMEM((SLOTW,), jnp.float32),
            pltpu.VMEM((GCH,), jnp.int32),
            pltpu.VMEM((GCH, DIM), jnp.float32),
            pltpu.VMEM((48,), jnp.int32),
            pltpu.VMEM_SHARED((NSUB, 16), jnp.int32),
            pltpu.VMEM_SHARED((NSUB, PADTOT), jnp.int32),
            pltpu.VMEM_SHARED((NSUB, PADTOT), jnp.float32),
            pltpu.SemaphoreType.DMA,
        ],
    )


def _gmm_body(gid_ref, x_ref, w1_ref, b1_ref, w2_ref, b2_ref, ws_ref, y_ref):
    del gid_ref
    xt = x_ref[...].astype(jnp.bfloat16)
    y = jnp.zeros((TM, DIM), jnp.float32)
    CHUNK = 1024
    for j0 in range(0, HID, CHUNK):
        a = jnp.dot(xt, w1_ref[0, :, j0:j0 + CHUNK],
                    preferred_element_type=jnp.float32) + b1_ref[0, 0, j0:j0 + CHUNK]
        g = jnp.dot(xt, w1_ref[0, :, HID + j0:HID + j0 + CHUNK],
                    preferred_element_type=jnp.float32) + b1_ref[0, 0, HID + j0:HID + j0 + CHUNK]
        act = (a * jax.nn.sigmoid(a)) * g
        y = y + jnp.dot(act.astype(jnp.bfloat16), w2_ref[0, j0:j0 + CHUNK, :],
                        preferred_element_type=jnp.float32)
    y_ref[...] = (y + b2_ref[0, 0, :]) * ws_ref[0]


def _gmm(gids, xdisp, w1b, b1r, w2b, b2r, ws3):
    grid_spec = pltpu.PrefetchScalarGridSpec(
        num_scalar_prefetch=1,
        grid=(GT,),
        in_specs=[
            pl.BlockSpec((TM, DIM), lambda i, g: (i, 0)),
            pl.BlockSpec((1, DIM, INNER), lambda i, g: (g[i], 0, 0)),
            pl.BlockSpec((1, 1, INNER), lambda i, g: (g[i], 0, 0)),
            pl.BlockSpec((1, HID, DIM), lambda i, g: (g[i], 0, 0)),
            pl.BlockSpec((1, 1, DIM), lambda i, g: (g[i], 0, 0)),
            pl.BlockSpec((1, TM, 1), lambda i, g: (i, 0, 0)),
        ],
        out_specs=pl.BlockSpec((TM, DIM), lambda i, g: (i, 0)),
    )
    return pl.pallas_call(
        _gmm_body,
        grid_spec=grid_spec,
        out_shape=jax.ShapeDtypeStruct((PADTOT, DIM), jnp.float32),
        compiler_params=pltpu.CompilerParams(
            dimension_semantics=("arbitrary",),
        ),
    )(gids, xdisp, w1b, b1r, w2b, b2r, ws3)


def _sc_combine_body(ydisp_hbm, invpos_hbm, ysort_hbm, ip_v, idx_v, rows_v, sem):
    wid = lax.axis_index("s") * 2 + lax.axis_index("c")
    r0 = wid * RW
    pltpu.sync_copy(invpos_hbm.at[pl.ds(r0, RW)], ip_v)

    def gather_step(g, _):
        def stage_step(j, _):
            idx_v[pl.ds(j * 16, 16)] = ip_v[pl.ds(g * GCH + j * 16, 16)]
            return 0

        lax.fori_loop(0, GCH // 16, stage_step, 0)
        pltpu.async_copy(ydisp_hbm.at[idx_v], rows_v, sem).wait()
        pltpu.sync_copy(rows_v, ysort_hbm.at[pl.ds(r0 + g * GCH, GCH)])
        return 0

    lax.fori_loop(0, RW // GCH, gather_step, 0)


@functools.cache
def _sc_combine():
    return pl.kernel(
        _sc_combine_body,
        out_type=jax.ShapeDtypeStruct((P, DIM), jnp.float32),
        mesh=_mesh(),
        compiler_params=pltpu.CompilerParams(needs_layout_passes=False),
        scratch_types=[
            pltpu.VMEM((RW,), jnp.int32),
            pltpu.VMEM((GCH,), jnp.int32),
            pltpu.VMEM((GCH, DIM), jnp.float32),
            pltpu.SemaphoreType.DMA,
        ],
    )


def _add_body(y_ref, o_ref):
    o_ref[...] = y_ref[:, 0, :] + y_ref[:, 1, :]


def _pair_add(ysort3):
    return pl.pallas_call(
        _add_body,
        grid=(8,),
        out_shape=jax.ShapeDtypeStruct((N, DIM), jnp.float32),
        in_specs=[pl.BlockSpec((N // 8, K, DIM), lambda i: (i, 0, 0))],
        out_specs=pl.BlockSpec((N // 8, DIM), lambda i: (i, 0)),
    )(ysort3)


@jax.jit
def kernel(x, router_W, router_b, W1, b1, W2, b2):
    xf = x.reshape(N, DIM)
    eids, wts = _router(xf, router_W, router_b)
    xdisp, wslot, invpos, gids48 = _sc_dispatch()(
        eids.reshape(P), wts.reshape(P), xf)
    ydisp = _gmm(gids48[:GT], xdisp,
                 W1.astype(jnp.bfloat16), b1.reshape(E, 1, INNER),
                 W2.astype(jnp.bfloat16), b2.reshape(E, 1, DIM),
                 wslot.reshape(GT, TM, 1))
    ysorted = _sc_combine()(ydisp, invpos)
    out = _pair_add(ysorted.reshape(N, K, DIM))
    return out.reshape(B, T, DIM)


# speedup vs baseline: 1.8362x; 1.4695x over previous
import functools

import jax
import jax.numpy as jnp
from jax import lax
from jax.experimental import pallas as pl
from jax.experimental.pallas import tpu as pltpu
from jax.experimental.pallas import tpu_sc as plsc

B, T, DIM = 2, 2048, 768
E, K, HID = 8, 2, 2048
INNER = 2 * HID
N = B * T
P = N * K
TM = 256
PADTOT = P + E * TM
GT = PADTOT // TM
NSUB = 16
NW = 32
CP = P // NW
GCH = 64
NB = 16
BP = P // NB


def _router_body(x_ref, rw_ref, rb_ref, eid_ref, w_ref):
    logits = jax.lax.dot_general(
        x_ref[...], rw_ref[...], (((1,), (1,)), ((), ())),
        preferred_element_type=jnp.float32,
    ) + rb_ref[...]
    idx = jax.lax.broadcasted_iota(jnp.int32, (N, E), 1)
    m1 = jnp.max(logits, axis=1, keepdims=True)
    a1 = jnp.min(jnp.where(logits == m1, idx, E), axis=1, keepdims=True)
    l2 = jnp.where(idx == a1, -jnp.inf, logits)
    m2 = jnp.max(l2, axis=1, keepdims=True)
    a2 = jnp.min(jnp.where(l2 == m2, idx, E), axis=1, keepdims=True)
    w1 = jax.nn.sigmoid(m1 - m2)
    eid_ref[...] = jnp.concatenate([a1, a2], axis=1)
    w_ref[...] = jnp.concatenate([w1, 1.0 - w1], axis=1)


def _router(xf, router_W, router_b):
    return pl.pallas_call(
        _router_body,
        out_shape=(
            jax.ShapeDtypeStruct((N, K), jnp.int32),
            jax.ShapeDtypeStruct((N, K), jnp.float32),
        ),
        in_specs=[
            pl.BlockSpec((N, DIM), lambda: (0, 0)),
            pl.BlockSpec((E, DIM), lambda: (0, 0)),
            pl.BlockSpec((E,), lambda: (0,)),
        ],
        out_specs=(
            pl.BlockSpec((N, K), lambda: (0, 0)),
            pl.BlockSpec((N, K), lambda: (0, 0)),
        ),
    )(xf, router_W, router_b)


def _bin_body(e_ref, pos_ref, gids_ref, tot_ref, run_ref):
    ph = pl.program_id(0)
    blk = pl.program_id(1)

    eb = e_ref[0]
    lanes8 = jax.lax.broadcasted_iota(jnp.int32, (1, E), 1)
    oh = (eb == lanes8).astype(jnp.float32)

    @pl.when(jnp.logical_and(ph == 0, blk == 0))
    def _init0():
        tot_ref[...] = jnp.zeros((1, 128), jnp.float32)

    @pl.when(ph == 0)
    def _count():
        tot_ref[:, :E] = tot_ref[:, :E] + jnp.sum(oh, axis=0, keepdims=True)
        pos_ref[...] = jnp.zeros((1, BP, 1), jnp.int32)

    @pl.when(jnp.logical_and(ph == 1, blk == 0))
    def _init1():
        run_ref[...] = jnp.zeros((1, 128), jnp.float32)

    @pl.when(ph == 1)
    def _rank():
        tot = tot_ref[:, :E]
        toti = tot.astype(jnp.int32)
        padcnt = ((toti + (TM - 1)) >> 8) << 8
        le8 = jax.lax.broadcasted_iota(jnp.int32, (E, E), 0) <= \
            jax.lax.broadcasted_iota(jnp.int32, (E, E), 1)
        incl = jax.lax.dot_general(
            padcnt.astype(jnp.float32), le8.astype(jnp.float32),
            (((1,), (0,)), ((), ())), preferred_element_type=jnp.float32)
        padoff = incl - padcnt.astype(jnp.float32)

        ir = jax.lax.broadcasted_iota(jnp.int32, (BP, BP), 0)
        ic = jax.lax.broadcasted_iota(jnp.int32, (BP, BP), 1)
        tri = (ic < ir).astype(jnp.float32)
        excl = jax.lax.dot_general(
            tri, oh, (((1,), (0,)), ((), ())),
            preferred_element_type=jnp.float32)

        slot = jnp.sum(oh * (padoff + run_ref[:, :E] + excl),
                       axis=1, keepdims=True)
        pos_ref[...] = slot.astype(jnp.int32).reshape(1, BP, 1)
        run_ref[:, :E] = run_ref[:, :E] + jnp.sum(oh, axis=0, keepdims=True)

        @pl.when(blk == 0)
        def _gids():
            ts = jax.lax.broadcasted_iota(jnp.int32, (1, 64), 1).astype(
                jnp.float32) * TM
            g = jnp.zeros((1, 64), jnp.float32)
            for e in range(E):
                incl_e = jax.lax.dot_general(
                    incl, (lanes8 == e).astype(jnp.float32),
                    (((1,), (1,)), ((), ())),
                    preferred_element_type=jnp.float32)
                g = g + jnp.where(ts >= incl_e, 1.0, 0.0)
            gids_ref[...] = jnp.minimum(g, E - 1).astype(jnp.int32)


def _binning(eT3):
    return pl.pallas_call(
        _bin_body,
        grid=(2, NB),
        out_shape=(
            jax.ShapeDtypeStruct((NB + 1, BP, 1), jnp.int32),
            jax.ShapeDtypeStruct((1, 64), jnp.int32),
        ),
        in_specs=[pl.BlockSpec((1, BP, 1), lambda ph, blk: (blk, 0, 0))],
        out_specs=(
            pl.BlockSpec((1, BP, 1),
                         lambda ph, blk: (jnp.where(ph == 0, NB, blk), 0, 0)),
            pl.BlockSpec((1, 64), lambda ph, blk: (0, 0)),
        ),
        scratch_shapes=[
            pltpu.VMEM((1, 128), jnp.float32),
            pltpu.VMEM((1, 128), jnp.float32),
        ],
        compiler_params=pltpu.CompilerParams(
            dimension_semantics=("arbitrary", "arbitrary"),
        ),
    )(eT3)


@functools.cache
def _mesh():
    return plsc.VectorSubcoreMesh(core_axis_name="c", subcore_axis_name="s",
                                  num_cores=2, num_subcores=NSUB)


def _sc_dispatch_body(pos_hbm, x_hbm, xdisp_hbm, pos_v, rows_v, sem):
    wid = lax.axis_index("s") * 2 + lax.axis_index("c")
    tokbase = (wid % NSUB) * CP
    pltpu.sync_copy(pos_hbm.at[pl.ds(wid * (CP // GCH), CP // GCH)], pos_v)

    def chunk_step(g, _):
        pltpu.sync_copy(x_hbm.at[pl.ds(tokbase + g * GCH, GCH)], rows_v)
        pltpu.async_copy(rows_v, xdisp_hbm.at[pos_v.at[g]], sem).wait()
        return 0

    lax.fori_loop(0, CP // GCH, chunk_step, 0)


@functools.cache
def _sc_dispatch():
    return pl.kernel(
        _sc_dispatch_body,
        out_type=jax.ShapeDtypeStruct((PADTOT, DIM), jnp.float32),
        mesh=_mesh(),
        compiler_params=pltpu.CompilerParams(needs_layout_passes=False),
        scratch_types=[
            pltpu.VMEM((CP // GCH, GCH), jnp.int32),
            pltpu.VMEM((GCH, DIM), jnp.float32),
            pltpu.SemaphoreType.DMA,
        ],
    )


def _gmm_body(gid_ref, x_ref, w1_ref, b1_ref, w2_ref, b2_ref, y_ref):
    del gid_ref
    xt = x_ref[...].astype(jnp.bfloat16)
    y = jnp.zeros((TM, DIM), jnp.float32)
    CHUNK = 1024
    for j0 in range(0, HID, CHUNK):
        a = jnp.dot(xt, w1_ref[0, :, j0:j0 + CHUNK],
                    preferred_element_type=jnp.float32) + b1_ref[0, 0, j0:j0 + CHUNK]
        g = jnp.dot(xt, w1_ref[0, :, HID + j0:HID + j0 + CHUNK],
                    preferred_element_type=jnp.float32) + b1_ref[0, 0, HID + j0:HID + j0 + CHUNK]
        act = (a * jax.nn.sigmoid(a)) * g
        y = y + jnp.dot(act.astype(jnp.bfloat16), w2_ref[0, j0:j0 + CHUNK, :],
                        preferred_element_type=jnp.float32)
    y_ref[...] = y + b2_ref[0, 0, :]


def _gmm(gids, xdisp, w1b, b1r, w2b, b2r):
    grid_spec = pltpu.PrefetchScalarGridSpec(
        num_scalar_prefetch=1,
        grid=(GT,),
        in_specs=[
            pl.BlockSpec((TM, DIM), lambda i, g: (i, 0)),
            pl.BlockSpec((1, DIM, INNER), lambda i, g: (g[i], 0, 0)),
            pl.BlockSpec((1, 1, INNER), lambda i, g: (g[i], 0, 0)),
            pl.BlockSpec((1, HID, DIM), lambda i, g: (g[i], 0, 0)),
            pl.BlockSpec((1, 1, DIM), lambda i, g: (g[i], 0, 0)),
        ],
        out_specs=pl.BlockSpec((TM, DIM), lambda i, g: (i, 0)),
    )
    return pl.pallas_call(
        _gmm_body,
        grid_spec=grid_spec,
        out_shape=jax.ShapeDtypeStruct((PADTOT, DIM), jnp.float32),
        compiler_params=pltpu.CompilerParams(
            dimension_semantics=("arbitrary",),
        ),
    )(gids, xdisp, w1b, b1r, w2b, b2r)


def _sc_combine_body(ydisp_hbm, pos_hbm, ysort_hbm, pos_v, rows_v, sem):
    wid = lax.axis_index("s") * 2 + lax.axis_index("c")
    r0 = wid * CP
    pltpu.sync_copy(pos_hbm.at[pl.ds(wid * (CP // GCH), CP // GCH)], pos_v)

    def chunk_step(g, _):
        pltpu.async_copy(ydisp_hbm.at[pos_v.at[g]], rows_v, sem).wait()
        pltpu.sync_copy(rows_v, ysort_hbm.at[pl.ds(r0 + g * GCH, GCH)])
        return 0

    lax.fori_loop(0, CP // GCH, chunk_step, 0)


@functools.cache
def _sc_combine():
    return pl.kernel(
        _sc_combine_body,
        out_type=jax.ShapeDtypeStruct((P, DIM), jnp.float32),
        mesh=_mesh(),
        compiler_params=pltpu.CompilerParams(needs_layout_passes=False),
        scratch_types=[
            pltpu.VMEM((CP // GCH, GCH), jnp.int32),
            pltpu.VMEM((GCH, DIM), jnp.float32),
            pltpu.SemaphoreType.DMA,
        ],
    )


def _add_body(w_ref, y0_ref, y1_ref, o_ref):
    w = w_ref[...]
    o_ref[...] = w[:, 0:1] * y0_ref[0] + w[:, 1:2] * y1_ref[0]


def _pair_add(wts, ys3):
    NTB = 8
    return pl.pallas_call(
        _add_body,
        grid=(NTB,),
        out_shape=jax.ShapeDtypeStruct((N, DIM), jnp.float32),
        in_specs=[
            pl.BlockSpec((N // NTB, K), lambda i: (i, 0)),
            pl.BlockSpec((1, N // NTB, DIM), lambda i: (0, i, 0)),
            pl.BlockSpec((1, N // NTB, DIM), lambda i: (1, i, 0)),
        ],
        out_specs=pl.BlockSpec((N // NTB, DIM), lambda i: (i, 0)),
    )(wts, ys3, ys3)


@jax.jit
def kernel(x, router_W, router_b, W1, b1, W2, b2):
    xf = x.reshape(N, DIM)
    eids, wts = _router(xf, router_W, router_b)
    eT3 = jnp.transpose(eids).reshape(NB, BP, 1)
    pos3, gids64 = _binning(eT3)
    posf = pos3[:NB].reshape(P)
    xdisp = _sc_dispatch()(posf.reshape(NW * (CP // GCH), GCH), xf)
    ydisp = _gmm(gids64.reshape(64)[:GT], xdisp,
                 W1.astype(jnp.bfloat16), b1.reshape(E, 1, INNER),
                 W2.astype(jnp.bfloat16), b2.reshape(E, 1, DIM))
    ysorted = _sc_combine()(ydisp, posf.reshape(NW * (CP // GCH), GCH))
    out = _pair_add(wts, ysorted.reshape(K, N, DIM))
    return out.reshape(B, T, DIM)


# speedup vs baseline: 2.1105x; 1.1494x over previous
import functools

import jax
import jax.numpy as jnp
from jax import lax
from jax.experimental import pallas as pl
from jax.experimental.pallas import tpu as pltpu
from jax.experimental.pallas import tpu_sc as plsc

B, T, DIM = 2, 2048, 768
E, K, HID = 8, 2, 2048
INNER = 2 * HID
N = B * T
P = N * K
TM = 256
PADTOT = P + E * TM
GT = PADTOT // TM
NSUB = 16
NW = 32
CP = P // NW
GCH = 64
NB = 16
BP = P // NB


def _router_body(x_ref, rw_ref, rb_ref, eid_ref, w_ref):
    logits = jax.lax.dot_general(
        x_ref[...], rw_ref[...], (((1,), (1,)), ((), ())),
        preferred_element_type=jnp.float32,
    ) + rb_ref[...]
    idx = jax.lax.broadcasted_iota(jnp.int32, (N, E), 1)
    m1 = jnp.max(logits, axis=1, keepdims=True)
    a1 = jnp.min(jnp.where(logits == m1, idx, E), axis=1, keepdims=True)
    l2 = jnp.where(idx == a1, -jnp.inf, logits)
    m2 = jnp.max(l2, axis=1, keepdims=True)
    a2 = jnp.min(jnp.where(l2 == m2, idx, E), axis=1, keepdims=True)
    w1 = jax.nn.sigmoid(m1 - m2)
    eid_ref[...] = jnp.concatenate([a1, a2], axis=1)
    w_ref[...] = jnp.concatenate([w1, 1.0 - w1], axis=1)


def _router(xf, router_W, router_b):
    return pl.pallas_call(
        _router_body,
        out_shape=(
            jax.ShapeDtypeStruct((N, K), jnp.int32),
            jax.ShapeDtypeStruct((N, K), jnp.float32),
        ),
        in_specs=[
            pl.BlockSpec((N, DIM), lambda: (0, 0)),
            pl.BlockSpec((E, DIM), lambda: (0, 0)),
            pl.BlockSpec((E,), lambda: (0,)),
        ],
        out_specs=(
            pl.BlockSpec((N, K), lambda: (0, 0)),
            pl.BlockSpec((N, K), lambda: (0, 0)),
        ),
    )(xf, router_W, router_b)


def _bin_body(e_ref, pos_ref, gids_ref, tot_ref, run_ref):
    ph = pl.program_id(0)
    blk = pl.program_id(1)

    eb = e_ref[0]
    lanes8 = jax.lax.broadcasted_iota(jnp.int32, (1, E), 1)
    oh = (eb == lanes8).astype(jnp.float32)

    @pl.when(jnp.logical_and(ph == 0, blk == 0))
    def _init0():
        tot_ref[...] = jnp.zeros((1, 128), jnp.float32)

    @pl.when(ph == 0)
    def _count():
        tot_ref[:, :E] = tot_ref[:, :E] + jnp.sum(oh, axis=0, keepdims=True)
        pos_ref[...] = jnp.zeros((1, BP, 1), jnp.int32)

    @pl.when(jnp.logical_and(ph == 1, blk == 0))
    def _init1():
        run_ref[...] = jnp.zeros((1, 128), jnp.float32)

    @pl.when(ph == 1)
    def _rank():
        tot = tot_ref[:, :E]
        toti = tot.astype(jnp.int32)
        padcnt = ((toti + (TM - 1)) >> 8) << 8
        le8 = jax.lax.broadcasted_iota(jnp.int32, (E, E), 0) <= \
            jax.lax.broadcasted_iota(jnp.int32, (E, E), 1)
        incl = jax.lax.dot_general(
            padcnt.astype(jnp.float32), le8.astype(jnp.float32),
            (((1,), (0,)), ((), ())), preferred_element_type=jnp.float32)
        padoff = incl - padcnt.astype(jnp.float32)

        ir = jax.lax.broadcasted_iota(jnp.int32, (BP, BP), 0)
        ic = jax.lax.broadcasted_iota(jnp.int32, (BP, BP), 1)
        tri = (ic < ir).astype(jnp.float32)
        excl = jax.lax.dot_general(
            tri, oh, (((1,), (0,)), ((), ())),
            preferred_element_type=jnp.float32)

        slot = jnp.sum(oh * (padoff + run_ref[:, :E] + excl),
                       axis=1, keepdims=True)
        pos_ref[...] = slot.astype(jnp.int32).reshape(1, BP, 1)
        run_ref[:, :E] = run_ref[:, :E] + jnp.sum(oh, axis=0, keepdims=True)

        @pl.when(blk == 0)
        def _gids():
            ts = jax.lax.broadcasted_iota(jnp.int32, (1, 64), 1).astype(
                jnp.float32) * TM
            g = jnp.zeros((1, 64), jnp.float32)
            for e in range(E):
                incl_e = jax.lax.dot_general(
                    incl, (lanes8 == e).astype(jnp.float32),
                    (((1,), (1,)), ((), ())),
                    preferred_element_type=jnp.float32)
                g = g + jnp.where(ts >= incl_e, 1.0, 0.0)
            gids_ref[...] = jnp.minimum(g, E - 1).astype(jnp.int32)


def _binning(eT3):
    return pl.pallas_call(
        _bin_body,
        grid=(2, NB),
        out_shape=(
            jax.ShapeDtypeStruct((NB + 1, BP, 1), jnp.int32),
            jax.ShapeDtypeStruct((1, 64), jnp.int32),
        ),
        in_specs=[pl.BlockSpec((1, BP, 1), lambda ph, blk: (blk, 0, 0))],
        out_specs=(
            pl.BlockSpec((1, BP, 1),
                         lambda ph, blk: (jnp.where(ph == 0, NB, blk), 0, 0)),
            pl.BlockSpec((1, 64), lambda ph, blk: (0, 0)),
        ),
        scratch_shapes=[
            pltpu.VMEM((1, 128), jnp.float32),
            pltpu.VMEM((1, 128), jnp.float32),
        ],
        compiler_params=pltpu.CompilerParams(
            dimension_semantics=("arbitrary", "arbitrary"),
        ),
    )(eT3)


@functools.cache
def _mesh():
    return plsc.VectorSubcoreMesh(core_axis_name="c", subcore_axis_name="s",
                                  num_cores=2, num_subcores=NSUB)


def _sc_dispatch_body(pos_hbm, x_hbm, xdisp_hbm, pos_v, rows_v, sem):
    wid = lax.axis_index("s") * 2 + lax.axis_index("c")
    tokbase = (wid % NSUB) * CP
    pltpu.sync_copy(pos_hbm.at[pl.ds(wid * (CP // GCH), CP // GCH)], pos_v)

    def chunk_step(g, _):
        pltpu.sync_copy(x_hbm.at[pl.ds(tokbase + g * GCH, GCH)], rows_v)
        pltpu.async_copy(rows_v, xdisp_hbm.at[pos_v.at[g]], sem).wait()
        return 0

    lax.fori_loop(0, CP // GCH, chunk_step, 0)


@functools.cache
def _sc_dispatch():
    return pl.kernel(
        _sc_dispatch_body,
        out_type=jax.ShapeDtypeStruct((PADTOT, DIM), jnp.float32),
        mesh=_mesh(),
        compiler_params=pltpu.CompilerParams(needs_layout_passes=False),
        scratch_types=[
            pltpu.VMEM((CP // GCH, GCH), jnp.int32),
            pltpu.VMEM((GCH, DIM), jnp.float32),
            pltpu.SemaphoreType.DMA,
        ],
    )


def _gmm_body(gid_ref, x_ref, w1_ref, b1_ref, w2_ref, b2_ref, y_ref):
    del gid_ref
    xt = x_ref[...]
    y = jnp.zeros((TM, DIM), jnp.float32)
    CHUNK = 1024
    for j0 in range(0, HID, CHUNK):
        a = jnp.dot(xt, w1_ref[0, :, j0:j0 + CHUNK],
                    preferred_element_type=jnp.float32) + b1_ref[0, 0, j0:j0 + CHUNK]
        g = jnp.dot(xt, w1_ref[0, :, HID + j0:HID + j0 + CHUNK],
                    preferred_element_type=jnp.float32) + b1_ref[0, 0, HID + j0:HID + j0 + CHUNK]
        act = (a * jax.nn.sigmoid(a)) * g
        y = y + jnp.dot(act, w2_ref[0, j0:j0 + CHUNK, :],
                        preferred_element_type=jnp.float32)
    y_ref[...] = y + b2_ref[0, 0, :]


def _gmm(gids, xdisp, w1b, b1r, w2b, b2r):
    grid_spec = pltpu.PrefetchScalarGridSpec(
        num_scalar_prefetch=1,
        grid=(GT,),
        in_specs=[
            pl.BlockSpec((TM, DIM), lambda i, g: (i, 0)),
            pl.BlockSpec((1, DIM, INNER), lambda i, g: (g[i], 0, 0)),
            pl.BlockSpec((1, 1, INNER), lambda i, g: (g[i], 0, 0)),
            pl.BlockSpec((1, HID, DIM), lambda i, g: (g[i], 0, 0)),
            pl.BlockSpec((1, 1, DIM), lambda i, g: (g[i], 0, 0)),
        ],
        out_specs=pl.BlockSpec((TM, DIM), lambda i, g: (i, 0)),
    )
    return pl.pallas_call(
        _gmm_body,
        grid_spec=grid_spec,
        out_shape=jax.ShapeDtypeStruct((PADTOT, DIM), jnp.float32),
        compiler_params=pltpu.CompilerParams(
            dimension_semantics=("arbitrary",),
        ),
    )(gids, xdisp, w1b, b1r, w2b, b2r)


def _sc_combine_body(ydisp_hbm, pos_hbm, ysort_hbm, pos_v, rows_v, sem):
    wid = lax.axis_index("s") * 2 + lax.axis_index("c")
    r0 = wid * CP
    pltpu.sync_copy(pos_hbm.at[pl.ds(wid * (CP // GCH), CP // GCH)], pos_v)

    def chunk_step(g, _):
        pltpu.async_copy(ydisp_hbm.at[pos_v.at[g]], rows_v, sem).wait()
        pltpu.sync_copy(rows_v, ysort_hbm.at[pl.ds(r0 + g * GCH, GCH)])
        return 0

    lax.fori_loop(0, CP // GCH, chunk_step, 0)


@functools.cache
def _sc_combine():
    return pl.kernel(
        _sc_combine_body,
        out_type=jax.ShapeDtypeStruct((P, DIM), jnp.float32),
        mesh=_mesh(),
        compiler_params=pltpu.CompilerParams(needs_layout_passes=False),
        scratch_types=[
            pltpu.VMEM((CP // GCH, GCH), jnp.int32),
            pltpu.VMEM((GCH, DIM), jnp.float32),
            pltpu.SemaphoreType.DMA,
        ],
    )


def _add_body(w_ref, y0_ref, y1_ref, o_ref):
    w = w_ref[...]
    o_ref[...] = w[:, 0:1] * y0_ref[0] + w[:, 1:2] * y1_ref[0]


def _pair_add(wts, ys3):
    NTB = 8
    return pl.pallas_call(
        _add_body,
        grid=(NTB,),
        out_shape=jax.ShapeDtypeStruct((N, DIM), jnp.float32),
        in_specs=[
            pl.BlockSpec((N // NTB, K), lambda i: (i, 0)),
            pl.BlockSpec((1, N // NTB, DIM), lambda i: (0, i, 0)),
            pl.BlockSpec((1, N // NTB, DIM), lambda i: (1, i, 0)),
        ],
        out_specs=pl.BlockSpec((N // NTB, DIM), lambda i: (i, 0)),
    )(wts, ys3, ys3)


@jax.jit
def kernel(x, router_W, router_b, W1, b1, W2, b2):
    xf = x.reshape(N, DIM)
    eids, wts = _router(xf, router_W, router_b)
    eT3 = jnp.transpose(eids).reshape(NB, BP, 1)
    pos3, gids64 = _binning(eT3)
    posf = pos3[:NB].reshape(P)
    xdisp = _sc_dispatch()(posf.reshape(NW * (CP // GCH), GCH), xf)
    ydisp = _gmm(gids64.reshape(64)[:GT], xdisp,
                 W1, b1.reshape(E, 1, INNER),
                 W2, b2.reshape(E, 1, DIM))
    ysorted = _sc_combine()(ydisp, posf.reshape(NW * (CP // GCH), GCH))
    out = _pair_add(wts, ysorted.reshape(K, N, DIM))
    return out.reshape(B, T, DIM)


# speedup vs baseline: 2.1758x; 1.0309x over previous
import functools

import jax
import jax.numpy as jnp
from jax import lax
from jax.experimental import pallas as pl
from jax.experimental.pallas import tpu as pltpu
from jax.experimental.pallas import tpu_sc as plsc

B, T, DIM = 2, 2048, 768
E, K, HID = 8, 2, 2048
INNER = 2 * HID
N = B * T
P = N * K
TM = 256
PADTOT = P + E * TM
GT = PADTOT // TM
NSUB = 16
NW = 32
CP = P // NW
GCH = 64
NB = 16
BP = P // NB


def _router_body(x_ref, rw_ref, rb_ref, eid_ref, w_ref):
    logits = jax.lax.dot_general(
        x_ref[...], rw_ref[...], (((1,), (1,)), ((), ())),
        preferred_element_type=jnp.float32,
    ) + rb_ref[...]
    idx = jax.lax.broadcasted_iota(jnp.int32, (N, E), 1)
    m1 = jnp.max(logits, axis=1, keepdims=True)
    a1 = jnp.min(jnp.where(logits == m1, idx, E), axis=1, keepdims=True)
    l2 = jnp.where(idx == a1, -jnp.inf, logits)
    m2 = jnp.max(l2, axis=1, keepdims=True)
    a2 = jnp.min(jnp.where(l2 == m2, idx, E), axis=1, keepdims=True)
    w1 = jax.nn.sigmoid(m1 - m2)
    eid_ref[...] = jnp.concatenate([a1, a2], axis=1)
    w_ref[...] = jnp.concatenate([w1, 1.0 - w1], axis=1)


def _router(xf, router_W, router_b):
    return pl.pallas_call(
        _router_body,
        out_shape=(
            jax.ShapeDtypeStruct((N, K), jnp.int32),
            jax.ShapeDtypeStruct((N, K), jnp.float32),
        ),
        in_specs=[
            pl.BlockSpec((N, DIM), lambda: (0, 0)),
            pl.BlockSpec((E, DIM), lambda: (0, 0)),
            pl.BlockSpec((E,), lambda: (0,)),
        ],
        out_specs=(
            pl.BlockSpec((N, K), lambda: (0, 0)),
            pl.BlockSpec((N, K), lambda: (0, 0)),
        ),
    )(xf, router_W, router_b)


def _bin_body(e_ref, pos_ref, gids_ref, tot_ref, run_ref):
    ph = pl.program_id(0)
    blk = pl.program_id(1)

    eb = e_ref[0]
    lanes8 = jax.lax.broadcasted_iota(jnp.int32, (1, E), 1)
    oh = (eb == lanes8).astype(jnp.float32)

    @pl.when(jnp.logical_and(ph == 0, blk == 0))
    def _init0():
        tot_ref[...] = jnp.zeros((1, 128), jnp.float32)

    @pl.when(ph == 0)
    def _count():
        tot_ref[:, :E] = tot_ref[:, :E] + jnp.sum(oh, axis=0, keepdims=True)
        pos_ref[...] = jnp.zeros((1, BP, 1), jnp.int32)

    @pl.when(jnp.logical_and(ph == 1, blk == 0))
    def _init1():
        run_ref[...] = jnp.zeros((1, 128), jnp.float32)

    @pl.when(ph == 1)
    def _rank():
        tot = tot_ref[:, :E]
        toti = tot.astype(jnp.int32)
        padcnt = ((toti + (TM - 1)) >> 8) << 8
        le8 = jax.lax.broadcasted_iota(jnp.int32, (E, E), 0) <= \
            jax.lax.broadcasted_iota(jnp.int32, (E, E), 1)
        incl = jax.lax.dot_general(
            padcnt.astype(jnp.float32), le8.astype(jnp.float32),
            (((1,), (0,)), ((), ())), preferred_element_type=jnp.float32)
        padoff = incl - padcnt.astype(jnp.float32)

        ir = jax.lax.broadcasted_iota(jnp.int32, (BP, BP), 0)
        ic = jax.lax.broadcasted_iota(jnp.int32, (BP, BP), 1)
        tri = (ic < ir).astype(jnp.float32)
        excl = jax.lax.dot_general(
            tri, oh, (((1,), (0,)), ((), ())),
            preferred_element_type=jnp.float32)

        slot = jnp.sum(oh * (padoff + run_ref[:, :E] + excl),
                       axis=1, keepdims=True)
        pos_ref[...] = slot.astype(jnp.int32).reshape(1, BP, 1)
        run_ref[:, :E] = run_ref[:, :E] + jnp.sum(oh, axis=0, keepdims=True)

        @pl.when(blk == 0)
        def _gids():
            ts = jax.lax.broadcasted_iota(jnp.int32, (1, 64), 1).astype(
                jnp.float32) * TM
            g = jnp.zeros((1, 64), jnp.float32)
            for e in range(E):
                incl_e = jax.lax.dot_general(
                    incl, (lanes8 == e).astype(jnp.float32),
                    (((1,), (1,)), ((), ())),
                    preferred_element_type=jnp.float32)
                g = g + jnp.where(ts >= incl_e, 1.0, 0.0)
            gids_ref[...] = jnp.minimum(g, E - 1).astype(jnp.int32)


def _binning(eT3):
    return pl.pallas_call(
        _bin_body,
        grid=(2, NB),
        out_shape=(
            jax.ShapeDtypeStruct((NB + 1, BP, 1), jnp.int32),
            jax.ShapeDtypeStruct((1, 64), jnp.int32),
        ),
        in_specs=[pl.BlockSpec((1, BP, 1), lambda ph, blk: (blk, 0, 0))],
        out_specs=(
            pl.BlockSpec((1, BP, 1),
                         lambda ph, blk: (jnp.where(ph == 0, NB, blk), 0, 0)),
            pl.BlockSpec((1, 64), lambda ph, blk: (0, 0)),
        ),
        scratch_shapes=[
            pltpu.VMEM((1, 128), jnp.float32),
            pltpu.VMEM((1, 128), jnp.float32),
        ],
        compiler_params=pltpu.CompilerParams(
            dimension_semantics=("arbitrary", "arbitrary"),
        ),
    )(eT3)


@functools.cache
def _mesh():
    return plsc.VectorSubcoreMesh(core_axis_name="c", subcore_axis_name="s",
                                  num_cores=2, num_subcores=NSUB)


def _sc_dispatch_body(pos_hbm, x_hbm, xdisp_hbm, pos_v, rows_v, sem):
    wid = lax.axis_index("s") * 2 + lax.axis_index("c")
    tokbase = (wid % NSUB) * CP
    pltpu.sync_copy(pos_hbm.at[pl.ds(wid * (CP // GCH), CP // GCH)], pos_v)

    def chunk_step(g, _):
        pltpu.sync_copy(x_hbm.at[pl.ds(tokbase + g * GCH, GCH)], rows_v)
        pltpu.async_copy(rows_v, xdisp_hbm.at[pos_v.at[g]], sem).wait()
        return 0

    lax.fori_loop(0, CP // GCH, chunk_step, 0)


@functools.cache
def _sc_dispatch():
    return pl.kernel(
        _sc_dispatch_body,
        out_type=jax.ShapeDtypeStruct((PADTOT, DIM), jnp.float32),
        mesh=_mesh(),
        compiler_params=pltpu.CompilerParams(needs_layout_passes=False),
        scratch_types=[
            pltpu.VMEM((CP // GCH, GCH), jnp.int32),
            pltpu.VMEM((GCH, DIM), jnp.float32),
            pltpu.SemaphoreType.DMA,
        ],
    )


def _gmm_body(gid_ref, x_ref, w1_ref, b1_ref, w2_ref, b2_ref, y_ref):
    del gid_ref
    xt = x_ref[...]
    y = jnp.zeros((TM, DIM), jnp.float32)
    CHUNK = 2048
    for j0 in range(0, HID, CHUNK):
        a = jnp.dot(xt, w1_ref[0, :, j0:j0 + CHUNK],
                    preferred_element_type=jnp.float32) + b1_ref[0, 0, j0:j0 + CHUNK]
        g = jnp.dot(xt, w1_ref[0, :, HID + j0:HID + j0 + CHUNK],
                    preferred_element_type=jnp.float32) + b1_ref[0, 0, HID + j0:HID + j0 + CHUNK]
        act = (a * jax.nn.sigmoid(a)) * g
        y = y + jnp.dot(act, w2_ref[0, j0:j0 + CHUNK, :],
                        preferred_element_type=jnp.float32)
    y_ref[...] = y + b2_ref[0, 0, :]


def _gmm(gids, xdisp, w1b, b1r, w2b, b2r):
    grid_spec = pltpu.PrefetchScalarGridSpec(
        num_scalar_prefetch=1,
        grid=(GT,),
        in_specs=[
            pl.BlockSpec((TM, DIM), lambda i, g: (i, 0)),
            pl.BlockSpec((1, DIM, INNER), lambda i, g: (g[i], 0, 0)),
            pl.BlockSpec((1, 1, INNER), lambda i, g: (g[i], 0, 0)),
            pl.BlockSpec((1, HID, DIM), lambda i, g: (g[i], 0, 0)),
            pl.BlockSpec((1, 1, DIM), lambda i, g: (g[i], 0, 0)),
        ],
        out_specs=pl.BlockSpec((TM, DIM), lambda i, g: (i, 0)),
    )
    return pl.pallas_call(
        _gmm_body,
        grid_spec=grid_spec,
        out_shape=jax.ShapeDtypeStruct((PADTOT, DIM), jnp.float32),
        compiler_params=pltpu.CompilerParams(
            dimension_semantics=("arbitrary",),
        ),
    )(gids, xdisp, w1b, b1r, w2b, b2r)


def _sc_combine_body(ydisp_hbm, pos_hbm, ysort_hbm, pos_v, rows_v, sem):
    wid = lax.axis_index("s") * 2 + lax.axis_index("c")
    r0 = wid * CP
    pltpu.sync_copy(pos_hbm.at[pl.ds(wid * (CP // GCH), CP // GCH)], pos_v)

    def chunk_step(g, _):
        pltpu.async_copy(ydisp_hbm.at[pos_v.at[g]], rows_v, sem).wait()
        pltpu.sync_copy(rows_v, ysort_hbm.at[pl.ds(r0 + g * GCH, GCH)])
        return 0

    lax.fori_loop(0, CP // GCH, chunk_step, 0)


@functools.cache
def _sc_combine():
    return pl.kernel(
        _sc_combine_body,
        out_type=jax.ShapeDtypeStruct((P, DIM), jnp.float32),
        mesh=_mesh(),
        compiler_params=pltpu.CompilerParams(needs_layout_passes=False),
        scratch_types=[
            pltpu.VMEM((CP // GCH, GCH), jnp.int32),
            pltpu.VMEM((GCH, DIM), jnp.float32),
            pltpu.SemaphoreType.DMA,
        ],
    )


def _add_body(w_ref, y0_ref, y1_ref, o_ref):
    w = w_ref[...]
    o_ref[...] = w[:, 0:1] * y0_ref[0] + w[:, 1:2] * y1_ref[0]


def _pair_add(wts, ys3):
    NTB = 8
    return pl.pallas_call(
        _add_body,
        grid=(NTB,),
        out_shape=jax.ShapeDtypeStruct((N, DIM), jnp.float32),
        in_specs=[
            pl.BlockSpec((N // NTB, K), lambda i: (i, 0)),
            pl.BlockSpec((1, N // NTB, DIM), lambda i: (0, i, 0)),
            pl.BlockSpec((1, N // NTB, DIM), lambda i: (1, i, 0)),
        ],
        out_specs=pl.BlockSpec((N // NTB, DIM), lambda i: (i, 0)),
    )(wts, ys3, ys3)


@jax.jit
def kernel(x, router_W, router_b, W1, b1, W2, b2):
    xf = x.reshape(N, DIM)
    eids, wts = _router(xf, router_W, router_b)
    eT3 = jnp.transpose(eids).reshape(NB, BP, 1)
    pos3, gids64 = _binning(eT3)
    posf = pos3[:NB].reshape(P)
    xdisp = _sc_dispatch()(posf.reshape(NW * (CP // GCH), GCH), xf)
    ydisp = _gmm(gids64.reshape(64)[:GT], xdisp,
                 W1, b1.reshape(E, 1, INNER),
                 W2, b2.reshape(E, 1, DIM))
    ysorted = _sc_combine()(ydisp, posf.reshape(NW * (CP // GCH), GCH))
    out = _pair_add(wts, ysorted.reshape(K, N, DIM))
    return out.reshape(B, T, DIM)
